# baseline (speedup 1.0000x reference)
import jax
import jax.numpy as jnp
from jax import lax
from jax.experimental import pallas as pl
from jax.experimental.pallas import tpu as pltpu
from jax.experimental.pallas import tpu_sc as plsc

_NC = 2
_NS = 16
_NW = _NC * _NS
_C = 128


def _make_segsum(d, npt, n_out_pad):
    rpt = n_out_pad // _NS
    hp = npt // 2
    mesh = plsc.VectorSubcoreMesh(core_axis_name="c", subcore_axis_name="s")

    def body(table, src, dst, zeros, out, idx_s, idx_d, rows0, rows1, acc,
             g0, g1, s0, s1):
        c = lax.axis_index("c")
        s = lax.axis_index("s")
        wid = s * _NC + c
        r0 = s * rpt

        def pair(k, carry):
            a = 2 * k
            b = a + 1
            nxt = lax.rem(a + 2, hp)
            pltpu.make_async_copy(table.at[idx_s.at[a]], rows0, g0).wait()
            pltpu.async_copy(table.at[idx_s.at[b]], rows1, g1)
            pltpu.async_copy(rows0, acc.at[idx_d.at[a]], s0, add=True)
            pltpu.make_async_copy(table.at[idx_s.at[b]], rows1, g1).wait()
            pltpu.make_async_copy(rows0, acc.at[idx_d.at[a]], s0).wait()
            pltpu.async_copy(rows1, acc.at[idx_d.at[b]], s1, add=True)
            pltpu.async_copy(table.at[idx_s.at[nxt]], rows0, g0)
            pltpu.make_async_copy(rows1, acc.at[idx_d.at[b]], s1).wait()
            return carry

        for p in range(2):
            pltpu.sync_copy(src.at[wid].at[pl.ds(p * hp, hp)], idx_s)
            pltpu.sync_copy(dst.at[wid].at[pl.ds(p * hp, hp)], idx_d)
            if p == 0:
                pltpu.sync_copy(zeros, acc.at[pl.ds(r0, rpt)])
            pltpu.async_copy(table.at[idx_s.at[0]], rows0, g0)
            if p == 0:
                plsc.subcore_barrier()
            lax.fori_loop(0, hp // 2, pair, 0)
            pltpu.make_async_copy(table.at[idx_s.at[0]], rows0, g0).wait()

        plsc.subcore_barrier()
        pltpu.sync_copy(acc.at[pl.ds(r0, rpt)], out.at[c].at[pl.ds(r0, rpt)])

    return pl.kernel(
        body,
        out_type=jax.ShapeDtypeStruct((_NC, n_out_pad, d), jnp.float32),
        mesh=mesh,
        scratch_types=[
            pltpu.VMEM((hp, _C), jnp.int32),
            pltpu.VMEM((hp, _C), jnp.int32),
            pltpu.VMEM((_C, d), jnp.float32),
            pltpu.VMEM((_C, d), jnp.float32),
            pltpu.VMEM_SHARED((n_out_pad, d), jnp.float32),
            pltpu.SemaphoreType.DMA,
            pltpu.SemaphoreType.DMA,
            pltpu.SemaphoreType.DMA,
            pltpu.SemaphoreType.DMA,
        ],
    )


def _segsum(table, src, dst, n_out):
    d = table.shape[1]
    e = src.shape[0]
    grp = _NW * _C * 4
    e_pad = -(-e // grp) * grp
    pad = e_pad - e
    if pad:
        src = jnp.concatenate([src, jnp.zeros((pad,), jnp.int32)])
        dst = jnp.concatenate([dst, n_out + (lax.iota(jnp.int32, pad) % 8)])
    n_out_pad = -(-(n_out + 8) // (_NS * 8)) * (_NS * 8)
    npt = e_pad // (_NW * _C)
    src2 = src.reshape(_NW, npt, _C)
    dst2 = dst.reshape(_NW, npt, _C)
    zeros = jnp.zeros((n_out_pad // _NS, d), jnp.float32)
    fn = _make_segsum(d, npt, n_out_pad)
    return fn(table, src2, dst2, zeros)


def _dense_layer(p, x, wn, ws, b, bn):
    n, din = x.shape
    dout = wn.shape[1]
    grid = -(-n // bn)

    def body(p_ref, x_ref, wn_ref, ws_ref, b_ref, o_ref):
        agg = p_ref[0] + p_ref[1]
        y = jnp.dot(agg, wn_ref[...], preferred_element_type=jnp.float32)
        y = y + jnp.dot(x_ref[...], ws_ref[...], preferred_element_type=jnp.float32)
        o_ref[...] = jnp.maximum(y + b_ref[...], 0.0)

    return pl.pallas_call(
        body,
        grid=(grid,),
        in_specs=[
            pl.BlockSpec((2, bn, din), lambda i: (0, i, 0)),
            pl.BlockSpec((bn, din), lambda i: (i, 0)),
            pl.BlockSpec((din, dout), lambda i: (0, 0)),
            pl.BlockSpec((din, dout), lambda i: (0, 0)),
            pl.BlockSpec((1, dout), lambda i: (0, 0)),
        ],
        out_specs=pl.BlockSpec((bn, dout), lambda i: (i, 0)),
        out_shape=jax.ShapeDtypeStruct((n, dout), jnp.float32),
    )(p, x, wn, ws, b.reshape(1, -1))


def _sum_partials(p):
    _, n_pad, dd = p.shape

    def body(p_ref, o_ref):
        o_ref[...] = p_ref[0] + p_ref[1]

    return pl.pallas_call(
        body,
        out_shape=jax.ShapeDtypeStruct((n_pad, dd), jnp.float32),
    )(p)


def _dense_layer_split(p_lo, p_hi, x_lo, x_hi, wn, ws, b):
    n, dd = x_lo.shape
    dout = wn.shape[1]

    def body(pl_ref, ph_ref, xl_ref, xh_ref, wnl_ref, wnh_ref, wsl_ref,
             wsh_ref, b_ref, o_ref):
        y = jnp.dot(pl_ref[0] + pl_ref[1], wnl_ref[...],
                    preferred_element_type=jnp.float32)
        y = y + jnp.dot(ph_ref[0] + ph_ref[1], wnh_ref[...],
                        preferred_element_type=jnp.float32)
        y = y + jnp.dot(xl_ref[...], wsl_ref[...],
                        preferred_element_type=jnp.float32)
        y = y + jnp.dot(xh_ref[...], wsh_ref[...],
                        preferred_element_type=jnp.float32)
        o_ref[...] = jnp.maximum(y + b_ref[...], 0.0)

    return pl.pallas_call(
        body,
        out_shape=jax.ShapeDtypeStruct((n, dout), jnp.float32),
    )(p_lo, p_hi, x_lo, x_hi, wn[:dd], wn[dd:], ws[:dd], ws[dd:],
      b.reshape(1, -1))


def _final_fused(p, xt1, wtn, wts, bt, batch_p, wm0, bm0, wm1, bm1, ng):
    n_pad = xt1.shape[0]
    ncls = wm1.shape[1]

    def body(p_ref, x_ref, wtn_ref, wts_ref, bt_ref, b_ref, wm0_ref, bm0_ref,
             wm1_ref, bm1_ref, o_ref):
        agg = p_ref[0] + p_ref[1]
        zt = jnp.dot(agg, wtn_ref[...], preferred_element_type=jnp.float32)
        zt = zt + jnp.dot(x_ref[...], wts_ref[...], preferred_element_type=jnp.float32)
        zt = jnp.maximum(zt + bt_ref[...], 0.0)
        gids = lax.broadcasted_iota(jnp.int32, (ng, n_pad), 0)
        oh = (gids == b_ref[...]).astype(jnp.float32)
        s = jnp.dot(oh, zt, preferred_element_type=jnp.float32)
        cnt = jnp.sum(oh, axis=1, keepdims=True)
        zg = s / jnp.maximum(cnt, 1.0)
        h = jnp.dot(zg, wm0_ref[...], preferred_element_type=jnp.float32)
        h = jnp.maximum(h + bm0_ref[...], 0.0)
        o_ref[...] = jnp.dot(h, wm1_ref[...], preferred_element_type=jnp.float32) + bm1_ref[...]

    return pl.pallas_call(
        body,
        out_shape=jax.ShapeDtypeStruct((ng, ncls), jnp.float32),
    )(p, xt1, wtn, wts, bt.reshape(1, -1), batch_p, wm0, bm0.reshape(1, -1),
      wm1, bm1.reshape(1, -1))


def kernel(x_cell, edge_index_cell, x_cell_batch, assignment, x_tissue,
           edge_index_tissue, x_tissue_batch,
           Wc_nbr_0, Wc_self_0, bc_0, Wc_nbr_1, Wc_self_1, bc_1,
           Wc_nbr_2, Wc_self_2, bc_2,
           Wt_nbr_0, Wt_self_0, bt_0, Wt_nbr_1, Wt_self_1, bt_1,
           Wm_0, bm_0, Wm_1, bm_1):
    n_cell, d = x_cell.shape
    n_tiss = x_tissue.shape[0]
    ng = 8

    src_c = edge_index_cell[0].astype(jnp.int32)
    dst_c = edge_index_cell[1].astype(jnp.int32)
    src_t = edge_index_tissue[0].astype(jnp.int32)
    dst_t = edge_index_tissue[1].astype(jnp.int32)

    x = x_cell
    for wn, ws, b in ((Wc_nbr_0, Wc_self_0, bc_0),
                      (Wc_nbr_1, Wc_self_1, bc_1),
                      (Wc_nbr_2, Wc_self_2, bc_2)):
        p = _segsum(x, src_c, dst_c, n_cell)
        x = _dense_layer(p, x, wn, ws, b, 512)

    pz = _segsum(x, lax.iota(jnp.int32, n_cell), assignment.astype(jnp.int32),
                 n_tiss)
    n_tp = pz.shape[1]
    pooled = _sum_partials(pz)
    xtp = jnp.zeros((n_tp, d), jnp.float32).at[:n_tiss].set(x_tissue)

    pt_lo = _segsum(pooled, src_t, dst_t, n_tiss)
    pt_hi = _segsum(xtp, src_t, dst_t, n_tiss)
    xt1 = _dense_layer_split(pt_lo, pt_hi, pooled, xtp, Wt_nbr_0, Wt_self_0,
                             bt_0)

    pt1 = _segsum(xt1, src_t, dst_t, n_tiss)
    batch_p = jnp.full((1, n_tp), ng + 1, jnp.int32).at[0, :n_tiss].set(
        x_tissue_batch.astype(jnp.int32))
    return _final_fused(pt1, xt1, Wt_nbr_1, Wt_self_1, bt_1, batch_p,
                        Wm_0, bm_0, Wm_1, bm_1, ng)


# speedup vs baseline: 3.1331x; 3.1331x over previous
import jax
import jax.numpy as jnp
from jax import lax
from jax.experimental import pallas as pl
from jax.experimental.pallas import tpu as pltpu
from jax.experimental.pallas import tpu_sc as plsc

_NC = 2
_NS = 16
_NW = _NC * _NS
_C = 128


def _make_segsum(d, npt, n_out_pad):
    rpt = n_out_pad // _NS
    hp = npt // 2
    mesh = plsc.VectorSubcoreMesh(core_axis_name="c", subcore_axis_name="s")

    def body(table, src, dst, zeros, out, idx_s, idx_d, rows0, rows1, acc,
             g0, g1, s0, s1):
        c = lax.axis_index("c")
        s = lax.axis_index("s")
        wid = s * _NC + c
        r0 = s * rpt

        def pair(k, carry):
            a = 2 * k
            b = a + 1
            nxt = lax.rem(a + 2, hp)
            pltpu.make_async_copy(table.at[idx_s.at[a]], rows0, g0).wait()
            pltpu.async_copy(table.at[idx_s.at[b]], rows1, g1)
            pltpu.async_copy(rows0, acc.at[idx_d.at[a]], s0, add=True)
            pltpu.make_async_copy(table.at[idx_s.at[b]], rows1, g1).wait()
            pltpu.make_async_copy(rows0, acc.at[idx_d.at[a]], s0).wait()
            pltpu.async_copy(rows1, acc.at[idx_d.at[b]], s1, add=True)
            pltpu.async_copy(table.at[idx_s.at[nxt]], rows0, g0)
            pltpu.make_async_copy(rows1, acc.at[idx_d.at[b]], s1).wait()
            return carry

        for p in range(2):
            pltpu.sync_copy(src.at[wid].at[pl.ds(p * hp, hp)], idx_s)
            pltpu.sync_copy(dst.at[wid].at[pl.ds(p * hp, hp)], idx_d)
            if p == 0:
                pltpu.sync_copy(zeros, acc.at[pl.ds(r0, rpt)])
            pltpu.async_copy(table.at[idx_s.at[0]], rows0, g0)
            if p == 0:
                plsc.subcore_barrier()
            lax.fori_loop(0, hp // 2, pair, 0)
            pltpu.make_async_copy(table.at[idx_s.at[0]], rows0, g0).wait()

        plsc.subcore_barrier()
        pltpu.sync_copy(acc.at[pl.ds(r0, rpt)], out.at[c].at[pl.ds(r0, rpt)])

    return pl.kernel(
        body,
        out_type=jax.ShapeDtypeStruct((_NC, n_out_pad, d), jnp.float32),
        mesh=mesh,
        scratch_types=[
            pltpu.VMEM((hp, _C), jnp.int32),
            pltpu.VMEM((hp, _C), jnp.int32),
            pltpu.VMEM((_C, d), jnp.float32),
            pltpu.VMEM((_C, d), jnp.float32),
            pltpu.VMEM_SHARED((n_out_pad, d), jnp.float32),
            pltpu.SemaphoreType.DMA,
            pltpu.SemaphoreType.DMA,
            pltpu.SemaphoreType.DMA,
            pltpu.SemaphoreType.DMA,
        ],
    )


def _segsum(table, src, dst, n_out, zrow):
    d = table.shape[1]
    e = src.shape[0]
    grp = _NW * _C * 4
    e_pad = -(-e // grp) * grp
    pad = e_pad - e
    if pad:
        ar = lax.iota(jnp.int32, pad)
        src = jnp.concatenate([src, zrow + (ar % 8)])
        dst = jnp.concatenate([dst, ar % n_out])
    n_out_pad = -(-(n_out + 8) // (_NS * 8)) * (_NS * 8)
    npt = e_pad // (_NW * _C)
    src2 = src.reshape(_NW, npt, _C)
    dst2 = dst.reshape(_NW, npt, _C)
    zeros = jnp.zeros((n_out_pad // _NS, d), jnp.float32)
    fn = _make_segsum(d, npt, n_out_pad)
    return fn(table, src2, dst2, zeros)


def _dense_layer(p, x, wn, ws, b, bn, n_real):
    n, din = x.shape
    dout = wn.shape[1]
    grid = -(-n // bn)

    def body(p_ref, x_ref, wn_ref, ws_ref, b_ref, o_ref):
        agg = p_ref[0] + p_ref[1]
        y = jnp.dot(agg, wn_ref[...], preferred_element_type=jnp.float32)
        y = y + jnp.dot(x_ref[...], ws_ref[...], preferred_element_type=jnp.float32)
        y = jnp.maximum(y + b_ref[...], 0.0)
        rid = pl.program_id(0) * bn + lax.broadcasted_iota(jnp.int32, y.shape, 0)
        o_ref[...] = jnp.where(rid < n_real, y, 0.0)

    return pl.pallas_call(
        body,
        grid=(grid,),
        in_specs=[
            pl.BlockSpec((2, bn, din), lambda i: (0, i, 0)),
            pl.BlockSpec((bn, din), lambda i: (i, 0)),
            pl.BlockSpec((din, dout), lambda i: (0, 0)),
            pl.BlockSpec((din, dout), lambda i: (0, 0)),
            pl.BlockSpec((1, dout), lambda i: (0, 0)),
        ],
        out_specs=pl.BlockSpec((bn, dout), lambda i: (i, 0)),
        out_shape=jax.ShapeDtypeStruct((n, dout), jnp.float32),
    )(p, x, wn, ws, b.reshape(1, -1))


def _sum_partials(p, n_real):
    _, n_pad, dd = p.shape

    def body(p_ref, o_ref):
        y = p_ref[0] + p_ref[1]
        rid = lax.broadcasted_iota(jnp.int32, y.shape, 0)
        o_ref[...] = jnp.where(rid < n_real, y, 0.0)

    return pl.pallas_call(
        body,
        out_shape=jax.ShapeDtypeStruct((n_pad, dd), jnp.float32),
    )(p)


def _dense_layer_split(p_lo, p_hi, x_lo, x_hi, wn, ws, b, n_real):
    n, dd = x_lo.shape
    dout = wn.shape[1]

    def body(pl_ref, ph_ref, xl_ref, xh_ref, wnl_ref, wnh_ref, wsl_ref,
             wsh_ref, b_ref, o_ref):
        y = jnp.dot(pl_ref[0] + pl_ref[1], wnl_ref[...],
                    preferred_element_type=jnp.float32)
        y = y + jnp.dot(ph_ref[0] + ph_ref[1], wnh_ref[...],
                        preferred_element_type=jnp.float32)
        y = y + jnp.dot(xl_ref[...], wsl_ref[...],
                        preferred_element_type=jnp.float32)
        y = y + jnp.dot(xh_ref[...], wsh_ref[...],
                        preferred_element_type=jnp.float32)
        y = jnp.maximum(y + b_ref[...], 0.0)
        rid = lax.broadcasted_iota(jnp.int32, y.shape, 0)
        o_ref[...] = jnp.where(rid < n_real, y, 0.0)

    return pl.pallas_call(
        body,
        out_shape=jax.ShapeDtypeStruct((n, dout), jnp.float32),
    )(p_lo, p_hi, x_lo, x_hi, wn[:dd], wn[dd:], ws[:dd], ws[dd:],
      b.reshape(1, -1))


def _final_fused(p, xt1, wtn, wts, bt, batch_p, wm0, bm0, wm1, bm1, ng):
    n_pad = xt1.shape[0]
    ncls = wm1.shape[1]

    def body(p_ref, x_ref, wtn_ref, wts_ref, bt_ref, b_ref, wm0_ref, bm0_ref,
             wm1_ref, bm1_ref, o_ref):
        agg = p_ref[0] + p_ref[1]
        zt = jnp.dot(agg, wtn_ref[...], preferred_element_type=jnp.float32)
        zt = zt + jnp.dot(x_ref[...], wts_ref[...], preferred_element_type=jnp.float32)
        zt = jnp.maximum(zt + bt_ref[...], 0.0)
        gids = lax.broadcasted_iota(jnp.int32, (ng, n_pad), 0)
        oh = (gids == b_ref[...]).astype(jnp.float32)
        s = jnp.dot(oh, zt, preferred_element_type=jnp.float32)
        cnt = jnp.sum(oh, axis=1, keepdims=True)
        zg = s / jnp.maximum(cnt, 1.0)
        h = jnp.dot(zg, wm0_ref[...], preferred_element_type=jnp.float32)
        h = jnp.maximum(h + bm0_ref[...], 0.0)
        o_ref[...] = jnp.dot(h, wm1_ref[...], preferred_element_type=jnp.float32) + bm1_ref[...]

    return pl.pallas_call(
        body,
        out_shape=jax.ShapeDtypeStruct((ng, ncls), jnp.float32),
    )(p, xt1, wtn, wts, bt.reshape(1, -1), batch_p, wm0, bm0.reshape(1, -1),
      wm1, bm1.reshape(1, -1))


def kernel(x_cell, edge_index_cell, x_cell_batch, assignment, x_tissue,
           edge_index_tissue, x_tissue_batch,
           Wc_nbr_0, Wc_self_0, bc_0, Wc_nbr_1, Wc_self_1, bc_1,
           Wc_nbr_2, Wc_self_2, bc_2,
           Wt_nbr_0, Wt_self_0, bt_0, Wt_nbr_1, Wt_self_1, bt_1,
           Wm_0, bm_0, Wm_1, bm_1):
    n_cell, d = x_cell.shape
    n_tiss = x_tissue.shape[0]
    ng = 8

    src_c = edge_index_cell[0].astype(jnp.int32)
    dst_c = edge_index_cell[1].astype(jnp.int32)
    src_t = edge_index_tissue[0].astype(jnp.int32)
    dst_t = edge_index_tissue[1].astype(jnp.int32)

    n_cp = n_cell + 16
    x = jnp.zeros((n_cp, d), jnp.float32).at[:n_cell].set(x_cell)
    for wn, ws, b in ((Wc_nbr_0, Wc_self_0, bc_0),
                      (Wc_nbr_1, Wc_self_1, bc_1),
                      (Wc_nbr_2, Wc_self_2, bc_2)):
        p = _segsum(x, src_c, dst_c, n_cell, n_cell)
        x = _dense_layer(p, x, wn, ws, b, 512, n_cell)

    pz = _segsum(x, lax.iota(jnp.int32, n_cell), assignment.astype(jnp.int32),
                 n_tiss, n_cell)
    n_tp = pz.shape[1]
    pooled = _sum_partials(pz, n_tiss)
    xtp = jnp.zeros((n_tp, d), jnp.float32).at[:n_tiss].set(x_tissue)

    pt_lo = _segsum(pooled, src_t, dst_t, n_tiss, n_tiss)
    pt_hi = _segsum(xtp, src_t, dst_t, n_tiss, n_tiss)
    xt1 = _dense_layer_split(pt_lo, pt_hi, pooled, xtp, Wt_nbr_0, Wt_self_0,
                             bt_0, n_tiss)

    pt1 = _segsum(xt1, src_t, dst_t, n_tiss, n_tiss)
    batch_p = jnp.full((1, n_tp), ng + 1, jnp.int32).at[0, :n_tiss].set(
        x_tissue_batch.astype(jnp.int32))
    return _final_fused(pt1, xt1, Wt_nbr_1, Wt_self_1, bt_1, batch_p,
                        Wm_0, bm_0, Wm_1, bm_1, ng)
